# baseline (speedup 1.0000x reference)
import functools

import jax
import jax.numpy as jnp
from jax import lax
from jax.experimental import pallas as pl
from jax.experimental.pallas import tpu as pltpu
from jax.experimental.pallas import tpu_sc as plsc


def _vq_body(n_total, d_dim, k_dim, x_ref, emb_ref, idx_ref, enc_ref,
             loss_ref, perp_ref, counts_ref, e2_ref, kb_ref, sse_ref):
    i = pl.program_id(0)
    nb = pl.num_programs(0)
    bn = x_ref.shape[0] * x_ref.shape[1]

    emb = emb_ref[...].reshape(k_dim, d_dim)

    @pl.when(i == 0)
    def _init():
        counts_ref[...] = jnp.zeros_like(counts_ref)
        e2_ref[...] = jnp.sum(emb * emb, axis=1)[None, :]
        kb_ref[...] = lax.bitcast_convert_type(
            lax.broadcasted_iota(jnp.int32, (1, k_dim), 1)
            | jnp.int32(0x3F800000), jnp.float32)
        sse_ref[0] = 0.0

    x = x_ref[...].reshape(bn, d_dim)
    x2 = jnp.sum(x * x, axis=1, keepdims=True)
    nxe2 = lax.dot_general(-2.0 * x, emb, (((1,), (1,)), ((), ())),
                           preferred_element_type=jnp.float32)
    ch = 128
    e2row = e2_ref[...]
    kbrow = kb_ref[...]
    sentinel = lax.bitcast_convert_type(jnp.int32(k_dim | 0x3F800000),
                                        jnp.float32)
    m = (x2 + nxe2[:, 0:ch]) + e2row[:, 0:ch]
    im = jnp.broadcast_to(kbrow[:, 0:ch], m.shape)
    for c in range(1, k_dim // ch):
        dc = (x2 + nxe2[:, c * ch:(c + 1) * ch]) + e2row[:, c * ch:(c + 1) * ch]
        kb_c = kbrow[:, c * ch:(c + 1) * ch]
        im = jnp.where(dc < m, kb_c, im)
        m = jnp.minimum(m, dc)
    dmin = jnp.min(m, axis=1, keepdims=True)
    qual = jnp.where(m == dmin, im, sentinel)
    idx = jnp.min(qual, axis=1, keepdims=True)
    enc = (kbrow == idx).astype(jnp.float32)
    enc_ref[...] = enc.reshape(bn, 1, k_dim)
    idx_ref[...] = lax.bitcast_convert_type(idx, jnp.int32) - jnp.int32(
        0x3F800000)

    counts_ref[...] += jnp.sum(enc, axis=0, keepdims=True)
    sse_ref[0] += jnp.sum(dmin)

    @pl.when(i == nb - 1)
    def _fini():
        loss_ref[0] = sse_ref[0] / (n_total * d_dim)
        avg = counts_ref[...] * (1.0 / n_total)
        perp_ref[0] = jnp.exp(-jnp.sum(avg * jnp.log(avg + 1e-10)))


def _sc_gather(table_hbm, idx_hbm, out_hbm, idx_v, rows_v, sem):
    b_per_w = idx_v.shape[0]
    wid = lax.axis_index("s") * 2 + lax.axis_index("c")
    base = wid * b_per_w
    pltpu.sync_copy(idx_hbm.at[pl.ds(base, b_per_w)], idx_v)
    pltpu.async_copy(table_hbm.at[idx_v], rows_v, sem).wait()
    pltpu.sync_copy(rows_v, out_hbm.at[pl.ds(base, b_per_w)])


def kernel(input, embedding):
    b, t, ld = input.shape
    l_dim, k_dim, d_dim = embedding.shape
    n = b * t * (ld // d_dim)
    gb = 4
    bn = (n // b) * gb
    grid = (b // gb,)

    idx, enc, loss, perp = pl.pallas_call(
        functools.partial(_vq_body, n, d_dim, k_dim),
        grid=grid,
        in_specs=[
            pl.BlockSpec((gb, t, ld), lambda i: (i, 0, 0)),
            pl.BlockSpec((l_dim, k_dim, d_dim), lambda i: (0, 0, 0)),
        ],
        out_specs=[
            pl.BlockSpec((bn, 1), lambda i: (i, 0)),
            pl.BlockSpec((bn, 1, k_dim), lambda i: (i, 0, 0)),
            pl.BlockSpec(memory_space=pltpu.SMEM),
            pl.BlockSpec(memory_space=pltpu.SMEM),
        ],
        out_shape=[
            jax.ShapeDtypeStruct((n, 1), jnp.int32),
            jax.ShapeDtypeStruct((n, l_dim, k_dim), jnp.float32),
            jax.ShapeDtypeStruct((1,), jnp.float32),
            jax.ShapeDtypeStruct((1,), jnp.float32),
        ],
        scratch_shapes=[
            pltpu.VMEM((1, k_dim), jnp.float32),
            pltpu.VMEM((1, k_dim), jnp.float32),
            pltpu.VMEM((1, k_dim), jnp.float32),
            pltpu.SMEM((1,), jnp.float32),
        ],
    )(input, embedding)

    nw = 32
    b_per_w = n // nw
    mesh = plsc.VectorSubcoreMesh(core_axis_name="c", subcore_axis_name="s")
    qst = functools.partial(
        pl.kernel, mesh=mesh,
        out_type=jax.ShapeDtypeStruct((n, d_dim), jnp.float32),
        scratch_types=[
            pltpu.VMEM((b_per_w,), jnp.int32),
            pltpu.VMEM((b_per_w, d_dim), jnp.float32),
            pltpu.SemaphoreType.DMA,
        ],
    )(_sc_gather)(embedding.reshape(k_dim, d_dim), idx.reshape(n))

    return qst.reshape(b, t, ld), enc, loss.reshape(()), perp.reshape(())


# speedup vs baseline: 2.2137x; 2.2137x over previous
import functools

import jax
import jax.numpy as jnp
from jax import lax
from jax.experimental import pallas as pl
from jax.experimental.pallas import tpu as pltpu


def _vq_body(n_total, d_dim, k_dim, x_ref, emb_ref, qst_ref, enc_ref,
             loss_ref, perp_ref, counts_ref, e2_ref, kb_ref, sse_ref):
    i = pl.program_id(0)
    nb = pl.num_programs(0)
    bn = x_ref.shape[0] * x_ref.shape[1]

    emb = emb_ref[...].reshape(k_dim, d_dim)

    @pl.when(i == 0)
    def _init():
        counts_ref[...] = jnp.zeros_like(counts_ref)
        e2_ref[...] = jnp.sum(emb * emb, axis=1)[None, :]
        kb_ref[...] = lax.bitcast_convert_type(
            lax.broadcasted_iota(jnp.int32, (1, k_dim), 1)
            | jnp.int32(0x3F800000), jnp.float32)
        sse_ref[0] = 0.0

    x = x_ref[...].reshape(bn, d_dim)
    x2 = jnp.sum(x * x, axis=1, keepdims=True)
    nxe2 = lax.dot_general(-2.0 * x, emb, (((1,), (1,)), ((), ())),
                           preferred_element_type=jnp.float32)
    ch = 128
    e2row = e2_ref[...]
    kbrow = kb_ref[...]
    sentinel = lax.bitcast_convert_type(jnp.int32(k_dim | 0x3F800000),
                                        jnp.float32)
    m = (x2 + nxe2[:, 0:ch]) + e2row[:, 0:ch]
    im = jnp.broadcast_to(kbrow[:, 0:ch], m.shape)
    for c in range(1, k_dim // ch):
        dc = (x2 + nxe2[:, c * ch:(c + 1) * ch]) + e2row[:, c * ch:(c + 1) * ch]
        kb_c = kbrow[:, c * ch:(c + 1) * ch]
        im = jnp.where(dc < m, kb_c, im)
        m = jnp.minimum(m, dc)
    dmin = jnp.min(m, axis=1, keepdims=True)
    qual = jnp.where(m == dmin, im, sentinel)
    idx = jnp.min(qual, axis=1, keepdims=True)
    enc = (kbrow == idx).astype(jnp.float32)
    enc_ref[...] = enc.reshape(bn, 1, k_dim)
    q = lax.dot_general(enc, emb, (((1,), (0,)), ((), ())),
                        preferred_element_type=jnp.float32)
    qst_ref[...] = q.reshape(x_ref.shape)

    counts_ref[...] += jnp.sum(enc, axis=0, keepdims=True)
    sse_ref[0] += jnp.sum(dmin)

    @pl.when(i == nb - 1)
    def _fini():
        loss_ref[0] = sse_ref[0] / (n_total * d_dim)
        avg = counts_ref[...] * (1.0 / n_total)
        perp_ref[0] = jnp.exp(-jnp.sum(avg * jnp.log(avg + 1e-10)))


def kernel(input, embedding):
    b, t, ld = input.shape
    l_dim, k_dim, d_dim = embedding.shape
    n = b * t * (ld // d_dim)
    gb = 4
    bn = (n // b) * gb
    grid = (b // gb,)

    qst, enc, loss, perp = pl.pallas_call(
        functools.partial(_vq_body, n, d_dim, k_dim),
        grid=grid,
        in_specs=[
            pl.BlockSpec((gb, t, ld), lambda i: (i, 0, 0)),
            pl.BlockSpec((l_dim, k_dim, d_dim), lambda i: (0, 0, 0)),
        ],
        out_specs=[
            pl.BlockSpec((gb, t, ld), lambda i: (i, 0, 0)),
            pl.BlockSpec((bn, 1, k_dim), lambda i: (i, 0, 0)),
            pl.BlockSpec(memory_space=pltpu.SMEM),
            pl.BlockSpec(memory_space=pltpu.SMEM),
        ],
        out_shape=[
            jax.ShapeDtypeStruct((b, t, ld), jnp.float32),
            jax.ShapeDtypeStruct((n, l_dim, k_dim), jnp.float32),
            jax.ShapeDtypeStruct((1,), jnp.float32),
            jax.ShapeDtypeStruct((1,), jnp.float32),
        ],
        scratch_shapes=[
            pltpu.VMEM((1, k_dim), jnp.float32),
            pltpu.VMEM((1, k_dim), jnp.float32),
            pltpu.VMEM((1, k_dim), jnp.float32),
            pltpu.SMEM((1,), jnp.float32),
        ],
    )(input, embedding)

    return qst, enc, loss.reshape(()), perp.reshape(())
